# baseline (speedup 1.0000x reference)
import jax
import jax.numpy as jnp
from jax import lax
from jax.experimental import pallas as pl
from jax.experimental.pallas import tpu as pltpu

B = 256
EMB = 16
K = 1024
HID = 32
GEN = 65536
RB = 64
CB = 32768
NRB = B // RB
NCB = GEN // CB


def _fused_kernel(z_ref, cb_ref, w1_ref, b1_ref, w2_ref, b2_ref, wk_ref,
                  out_ref, loss_ref, enc_s):
    j = pl.program_id(0)
    i = pl.program_id(1)

    @pl.when((j == 0) & (i == 0))
    def _vq_trunk():
        z = z_ref[...]
        cb = cb_ref[...]
        z2 = jnp.sum(z * z, axis=1, keepdims=True)
        cb2 = jnp.sum(cb * cb, axis=1, keepdims=True)
        cross = lax.dot_general(z, cb, (((1,), (1,)), ((), ())),
                                preferred_element_type=jnp.float32)
        d = z2 - 2.0 * cross + cb2.T
        dmin = jnp.min(d, axis=1, keepdims=True)
        ii = lax.broadcasted_iota(jnp.int32, d.shape, 1)
        idx = jnp.min(jnp.where(d == dmin, ii, jnp.int32(K)), axis=1,
                      keepdims=True)
        onehot = (ii == idx).astype(jnp.float32)
        q = lax.dot_general(onehot, cb, (((1,), (0,)), ((), ())),
                            preferred_element_type=jnp.float32)
        diff = q - z
        loss_ref[0, 0] = 1.25 * jnp.sum(diff * diff) / (B * EMB)
        h = lax.dot_general(q, w1_ref[...], (((1,), (1,)), ((), ())),
                            preferred_element_type=jnp.float32) + b1_ref[...]
        h = jnp.maximum(h, 0.0)
        enc_s[...] = lax.dot_general(h, w2_ref[...], (((1,), (1,)), ((), ())),
                                     preferred_element_type=jnp.float32) \
            + b2_ref[...]

    enc_rows = enc_s[pl.ds(i * RB, RB), :]
    out_ref[...] = lax.dot_general(
        enc_rows, wk_ref[...], (((1,), (0,)), ((), ())),
        preferred_element_type=jnp.float32)


@jax.jit
def kernel(z, codebook, W1, b1, W2, b2, Wk):
    gen, loss = pl.pallas_call(
        _fused_kernel,
        grid=(NCB, NRB),
        out_shape=(
            jax.ShapeDtypeStruct((B, GEN), jnp.float32),
            jax.ShapeDtypeStruct((1, 1), jnp.float32),
        ),
        in_specs=[
            pl.BlockSpec((B, EMB), lambda j, i: (0, 0)),
            pl.BlockSpec((K, EMB), lambda j, i: (0, 0)),
            pl.BlockSpec((HID, EMB), lambda j, i: (0, 0)),
            pl.BlockSpec((1, HID), lambda j, i: (0, 0)),
            pl.BlockSpec((EMB, HID), lambda j, i: (0, 0)),
            pl.BlockSpec((1, EMB), lambda j, i: (0, 0)),
            pl.BlockSpec((EMB, CB), lambda j, i: (0, j)),
        ],
        out_specs=(
            pl.BlockSpec((RB, CB), lambda j, i: (i, j)),
            pl.BlockSpec(memory_space=pltpu.SMEM),
        ),
        scratch_shapes=[pltpu.VMEM((B, EMB), jnp.float32)],
        compiler_params=pltpu.CompilerParams(
            dimension_semantics=("arbitrary", "arbitrary")),
    )(z, codebook, W1, b1.reshape(1, HID), W2, b2.reshape(1, EMB), Wk)
    return gen, loss[0, 0]


# speedup vs baseline: 1.0339x; 1.0339x over previous
import jax
import jax.numpy as jnp
from jax import lax
from jax.experimental import pallas as pl
from jax.experimental.pallas import tpu as pltpu

B = 256
EMB = 16
K = 1024
HID = 32
GEN = 65536
CB = 8192
NCB = GEN // CB


def _fused_kernel(z_ref, cb_ref, w1_ref, b1_ref, w2_ref, b2_ref, wk_ref,
                  out_ref, loss_ref, enc_s):
    j = pl.program_id(0)

    @pl.when(j == 0)
    def _vq_trunk():
        z = z_ref[...]
        cb = cb_ref[...]
        hT = lax.dot_general(cb, w1_ref[...], (((1,), (1,)), ((), ())),
                             preferred_element_type=jnp.float32) + b1_ref[...]
        hT = jnp.maximum(hT, 0.0)
        encT = lax.dot_general(hT, w2_ref[...], (((1,), (1,)), ((), ())),
                               preferred_element_type=jnp.float32) \
            + b2_ref[...]
        z2 = jnp.sum(z * z, axis=1, keepdims=True)
        cb2 = jnp.sum(cb * cb, axis=1, keepdims=True)
        cross = lax.dot_general(z, cb, (((1,), (1,)), ((), ())),
                                preferred_element_type=jnp.float32)
        d = z2 - 2.0 * cross + cb2.T
        dmin = jnp.min(d, axis=1, keepdims=True)
        ii = lax.broadcasted_iota(jnp.int32, d.shape, 1)
        idx = jnp.min(jnp.where(d == dmin, ii, jnp.int32(K)), axis=1,
                      keepdims=True)
        loss_ref[0, 0] = 1.25 * jnp.sum(dmin) / (B * EMB)
        onehot = (ii == idx).astype(jnp.float32)
        enc_s[...] = lax.dot_general(onehot, encT, (((1,), (0,)), ((), ())),
                                     preferred_element_type=jnp.float32)

    out_ref[...] = lax.dot_general(
        enc_s[...], wk_ref[...], (((1,), (0,)), ((), ())),
        preferred_element_type=jnp.float32)


@jax.jit
def kernel(z, codebook, W1, b1, W2, b2, Wk):
    gen, loss = pl.pallas_call(
        _fused_kernel,
        grid=(NCB,),
        out_shape=(
            jax.ShapeDtypeStruct((B, GEN), jnp.float32),
            jax.ShapeDtypeStruct((1, 1), jnp.float32),
        ),
        in_specs=[
            pl.BlockSpec((B, EMB), lambda j: (0, 0)),
            pl.BlockSpec((K, EMB), lambda j: (0, 0)),
            pl.BlockSpec((HID, EMB), lambda j: (0, 0)),
            pl.BlockSpec((1, HID), lambda j: (0, 0)),
            pl.BlockSpec((EMB, HID), lambda j: (0, 0)),
            pl.BlockSpec((1, EMB), lambda j: (0, 0)),
            pl.BlockSpec((EMB, CB), lambda j: (0, j)),
        ],
        out_specs=(
            pl.BlockSpec((B, CB), lambda j: (0, j)),
            pl.BlockSpec(memory_space=pltpu.SMEM),
        ),
        scratch_shapes=[pltpu.VMEM((B, EMB), jnp.float32)],
        compiler_params=pltpu.CompilerParams(
            dimension_semantics=("arbitrary",)),
    )(z, codebook, W1, b1.reshape(1, HID), W2, b2.reshape(1, EMB), Wk)
    return gen, loss[0, 0]
